# baseline (speedup 1.0000x reference)
import jax
import jax.numpy as jnp
from jax import lax
from jax.experimental import pallas as pl
from jax.experimental.pallas import tpu as pltpu
from jax.experimental.pallas import tpu_sc as plsc

N = 8000
NROW = 8064
D = 128
O = 64
E = 120000
P = 200000

NW = 32
ECT = 60
EC0 = 24
EC1 = ECT - EC0
CB = 128
EPAD = 16 * ECT * CB
PC = 49
P_PER = PC * CB

BLK = NROW // 16

_SC_PARAMS = pltpu.CompilerParams(
    needs_layout_passes=False, use_tc_tiling_on_sc=False)


def _encode_body(xc, xd, wr00, ws0c, ws0d, wr10, ws1c, relv,
                 y0, y1, gc, p0d):
    f32 = jnp.float32
    xcb = xc[...]
    y0[...] = jnp.dot(xcb, wr00[...], preferred_element_type=f32)
    t = jnp.maximum(jnp.dot(xcb, ws0c[...], preferred_element_type=f32), 0.0)
    y1p = jnp.dot(t, wr10[...], preferred_element_type=f32)
    col = lax.broadcasted_iota(jnp.int32, (BLK, O), 1)
    ones_col = jnp.where(col == 0, 1.0, 0.0).astype(f32)
    y1[...] = jnp.concatenate([y1p, ones_col], axis=1)
    gc[...] = jnp.dot(t, ws1c[...], preferred_element_type=f32) * relv[...]
    p0d[...] = jnp.dot(xd[...], ws0d[...], preferred_element_type=f32)


def _encode(xc, xd, wr00, ws0c, ws0d, wr10, ws1c, relv):
    row = lambda i: (i, 0)
    full = lambda i: (0, 0)
    return pl.pallas_call(
        _encode_body,
        grid=(NROW // BLK,),
        in_specs=[
            pl.BlockSpec((BLK, D), row),
            pl.BlockSpec((BLK, D), row),
            pl.BlockSpec((D, D), full),
            pl.BlockSpec((D, D), full),
            pl.BlockSpec((D, D), full),
            pl.BlockSpec((D, O), full),
            pl.BlockSpec((D, O), full),
            pl.BlockSpec((1, O), full),
        ],
        out_specs=[
            pl.BlockSpec((BLK, D), row),
            pl.BlockSpec((BLK, D), row),
            pl.BlockSpec((BLK, O), row),
            pl.BlockSpec((BLK, D), row),
        ],
        out_shape=[
            jax.ShapeDtypeStruct((NROW, D), jnp.float32),
            jax.ShapeDtypeStruct((NROW, D), jnp.float32),
            jax.ShapeDtypeStruct((NROW, O), jnp.float32),
            jax.ShapeDtypeStruct((NROW, D), jnp.float32),
        ],
    )(xc, xd, wr00, ws0c, ws0d, wr10, ws1c, relv)


def _segsum_body(y0_hbm, y1_hbm, src_hbm, dst_hbm, z_hbm,
                 agg0_out, agg1_out,
                 src_v, dst_v, r0, r1,
                 agg_s, sem0, sem1):
    cid = lax.axis_index("c")
    sid = lax.axis_index("s")
    rbase = sid * BLK

    pltpu.sync_copy(z_hbm.at[pl.ds(rbase, BLK)], agg_s.at[pl.ds(rbase, BLK)])
    plsc.subcore_barrier()

    def run(kc, coff):
        pltpu.sync_copy(src_hbm.at[sid, pl.ds(coff, kc)],
                        src_v.at[pl.ds(0, kc)])
        pltpu.sync_copy(dst_hbm.at[sid, pl.ds(coff, kc)],
                        dst_v.at[pl.ds(0, kc)])

        def phase(y_hbm, agg_out):
            def start(c, rv, sem):
                pltpu.async_copy(y_hbm.at[src_v.at[c]], rv, sem)

            def wait(c, rv, sem):
                pltpu.make_async_copy(y_hbm.at[src_v.at[c]], rv, sem).wait()

            def scat(c, rv):
                pltpu.sync_copy(rv, agg_s.at[dst_v.at[c]], add=True)

            start(0, r0, sem0)

            def kstep(k, _):
                e = 2 * k
                o = e + 1
                start(o, r1, sem1)
                wait(e, r0, sem0)
                scat(e, r0)
                start(e + 2, r0, sem0)
                wait(o, r1, sem1)
                scat(o, r1)
                return _

            lax.fori_loop(0, (kc - 2) // 2, kstep, None)
            start(kc - 1, r1, sem1)
            wait(kc - 2, r0, sem0)
            scat(kc - 2, r0)
            wait(kc - 1, r1, sem1)
            scat(kc - 1, r1)
            plsc.subcore_barrier()
            pltpu.sync_copy(agg_s.at[pl.ds(rbase, BLK)],
                            agg_out.at[cid, pl.ds(rbase, BLK)])

        phase(y0_hbm, agg0_out)
        pltpu.sync_copy(z_hbm.at[pl.ds(rbase, BLK)],
                        agg_s.at[pl.ds(rbase, BLK)])
        plsc.subcore_barrier()
        phase(y1_hbm, agg1_out)

    @pl.when(cid == 0)
    def _core0():
        run(EC0, 0)

    @pl.when(cid == 1)
    def _core1():
        run(EC1, EC0)


def _segsum(y0, y1a, srcb, dstb, zblk):
    mesh = plsc.VectorSubcoreMesh(core_axis_name="c", subcore_axis_name="s")
    fn = pl.kernel(
        _segsum_body,
        out_type=[
            jax.ShapeDtypeStruct((2, NROW, D), jnp.float32),
            jax.ShapeDtypeStruct((2, NROW, D), jnp.float32),
        ],
        mesh=mesh,
        scratch_types=[
            pltpu.VMEM((EC1, CB), jnp.int32),
            pltpu.VMEM((EC1, CB), jnp.int32),
            pltpu.VMEM((CB, D), jnp.float32),
            pltpu.VMEM((CB, D), jnp.float32),
            pltpu.VMEM_SHARED((NROW, D), jnp.float32),
            pltpu.SemaphoreType.DMA,
            pltpu.SemaphoreType.DMA,
        ],
    )
    return fn(y0, y1a, srcb, dstb, zblk)


def _combine_body(p0d, a0a, a0b, a1a, a1b, ws1d, hd):
    col = lax.broadcasted_iota(jnp.int32, (BLK, D), 1)
    a0 = a0a[...] + a0b[...]
    a1 = a1a[...] + a1b[...]
    deg = jnp.sum(jnp.where(col == O, a1, 0.0), axis=1, keepdims=True)
    dd = jnp.maximum(deg, 1.0)
    h1d = jnp.maximum(p0d[...] + a0 / dd, 0.0)
    hd[...] = (jnp.dot(h1d, ws1d[...], preferred_element_type=jnp.float32)
               + a1[:, :O] / dd)


def _combine(p0d, a0a, a0b, a1a, a1b, ws1d):
    row = lambda i: (i, 0)
    full = lambda i: (0, 0)
    return pl.pallas_call(
        _combine_body,
        grid=(NROW // BLK,),
        in_specs=[
            pl.BlockSpec((BLK, D), row),
            pl.BlockSpec((BLK, D), row),
            pl.BlockSpec((BLK, D), row),
            pl.BlockSpec((BLK, D), row),
            pl.BlockSpec((BLK, D), row),
            pl.BlockSpec((D, O), full),
        ],
        out_specs=pl.BlockSpec((BLK, O), row),
        out_shape=jax.ShapeDtypeStruct((NROW, O), jnp.float32),
    )(p0d, a0a, a0b, a1a, a1b, ws1d)


def _decode_body(gc_hbm, hd_hbm, aidx_hbm, bidx_hbm, out_hbm,
                 aidx_v, bidx_v, gav, gbv, sc_v,
                 sav, sbv):
    cid = lax.axis_index("c")
    sid = lax.axis_index("s")
    wid = sid * 2 + cid
    pltpu.sync_copy(aidx_hbm.at[wid], aidx_v)
    pltpu.sync_copy(bidx_hbm.at[wid], bidx_v)

    lanes = lax.iota(jnp.int32, 16)

    def compute(cc, ga, gb):
        def group(g, _):
            pv = g * 16 + lanes
            acc = jnp.zeros((16,), jnp.float32)
            for j in range(O):
                jf = (lanes + j) & (O - 1)
                va = plsc.load_gather(ga, [pv, jf])
                vb = plsc.load_gather(gb, [pv, jf])
                acc = acc + va * vb
            sc_v[cc, pl.ds(g * 16, 16)] = acc
            return _

        lax.fori_loop(0, CB // 16, group, None)

    NB = 4

    def start(cc, b):
        pltpu.async_copy(gc_hbm.at[aidx_v.at[cc]], gav.at[b], sav.at[b])
        pltpu.async_copy(hd_hbm.at[bidx_v.at[cc]], gbv.at[b], sbv.at[b])

    def wait(cc, b):
        pltpu.make_async_copy(
            gc_hbm.at[aidx_v.at[cc]], gav.at[b], sav.at[b]).wait()
        pltpu.make_async_copy(
            hd_hbm.at[bidx_v.at[cc]], gbv.at[b], sbv.at[b]).wait()

    for b in range(NB):
        start(b, b)

    def kstep(k, _):
        for b in range(NB):
            cc = NB * k + b
            wait(cc, b)
            compute(cc, gav.at[b], gbv.at[b])

            @pl.when(cc + NB < PC)
            def _prefetch():
                start(cc + NB, b)

        return _

    lax.fori_loop(0, PC // NB, kstep, None)
    wait(PC - 1, (PC - 1) % NB)
    compute(PC - 1, gav.at[(PC - 1) % NB], gbv.at[(PC - 1) % NB])
    pltpu.sync_copy(sc_v, out_hbm.at[wid])


def _decode(gc, hd, aidxb, bidxb):
    mesh = plsc.VectorSubcoreMesh(core_axis_name="c", subcore_axis_name="s")
    fn = pl.kernel(
        _decode_body,
        out_type=jax.ShapeDtypeStruct((NW, PC, CB), jnp.float32),
        mesh=mesh,
        compiler_params=_SC_PARAMS,
        scratch_types=[
            pltpu.VMEM((PC, CB), jnp.int32),
            pltpu.VMEM((PC, CB), jnp.int32),
            pltpu.VMEM((4, CB, O), jnp.float32),
            pltpu.VMEM((4, CB, O), jnp.float32),
            pltpu.VMEM((PC, CB), jnp.float32),
            pltpu.SemaphoreType.DMA((4,)),
            pltpu.SemaphoreType.DMA((4,)),
        ],
    )
    return fn(gc, hd, aidxb, bidxb)


def kernel(x_Compound, x_Disease, x_Gene, x_Anatomy,
           edge_index_0, edge_index_1, edge_index_2, edge_index_3,
           W_self_0_Compound, W_self_0_Disease, W_self_0_Gene, W_self_0_Anatomy,
           W_rel_0_0, W_rel_0_1, W_rel_0_2, W_rel_0_3,
           W_self_1_Compound, W_self_1_Disease, W_self_1_Gene, W_self_1_Anatomy,
           W_rel_1_0, W_rel_1_1, W_rel_1_2, W_rel_1_3,
           rel_vec, edge_label_index):
    f32 = jnp.float32
    xc = jnp.pad(x_Compound.astype(f32), ((0, NROW - N), (0, 0)))
    xd = jnp.pad(x_Disease.astype(f32), ((0, NROW - N), (0, 0)))

    src = edge_index_0[0].astype(jnp.int32)
    dst = edge_index_0[1].astype(jnp.int32)
    epad = EPAD - E
    srcb = jnp.concatenate([src, jnp.zeros((epad,), jnp.int32)]).reshape(16, ECT, CB)
    dstb = jnp.concatenate([dst, jnp.full((epad,), N, jnp.int32)]).reshape(16, ECT, CB)

    eli = edge_label_index.astype(jnp.int32)
    ppad = NW * P_PER - P
    aidxb = jnp.concatenate([eli[0], jnp.zeros((ppad,), jnp.int32)]).reshape(NW, PC, CB)
    bidxb = jnp.concatenate([eli[1], jnp.zeros((ppad,), jnp.int32)]).reshape(NW, PC, CB)

    z128 = jnp.zeros((NROW, D), f32)

    y0, y1a, gc, p0d = _encode(
        xc, xd, W_rel_0_0, W_self_0_Compound, W_self_0_Disease,
        W_rel_1_0, W_self_1_Compound, rel_vec.reshape(1, O))

    agg0p, agg1p = _segsum(y0, y1a, srcb, dstb, z128)

    hd = _combine(p0d, agg0p[0], agg0p[1], agg1p[0], agg1p[1],
                  W_self_1_Disease)

    scores = _decode(gc, hd, aidxb, bidxb)
    return scores.reshape(-1)[:P]


# speedup vs baseline: 1.0314x; 1.0314x over previous
import jax
import jax.numpy as jnp
from jax import lax
from jax.experimental import pallas as pl
from jax.experimental.pallas import tpu as pltpu
from jax.experimental.pallas import tpu_sc as plsc

N = 8000
NROW = 8064
D = 128
O = 64
E = 120000
P = 200000

NW = 32
ECT = 60
EC0 = 36
EC1 = ECT - EC0
CB = 128
EPAD = 16 * ECT * CB
PC = 49
P_PER = PC * CB

BLK = NROW // 16

_SC_PARAMS = pltpu.CompilerParams(
    needs_layout_passes=False, use_tc_tiling_on_sc=False)


def _encode_body(xc, xd, wr00, ws0c, ws0d, wr10, ws1c, relv,
                 y0, y1, gc, p0d):
    f32 = jnp.float32
    xcb = xc[...]
    y0[...] = jnp.dot(xcb, wr00[...], preferred_element_type=f32)
    t = jnp.maximum(jnp.dot(xcb, ws0c[...], preferred_element_type=f32), 0.0)
    y1p = jnp.dot(t, wr10[...], preferred_element_type=f32)
    col = lax.broadcasted_iota(jnp.int32, (BLK, O), 1)
    ones_col = jnp.where(col == 0, 1.0, 0.0).astype(f32)
    y1[...] = jnp.concatenate([y1p, ones_col], axis=1)
    gc[...] = jnp.dot(t, ws1c[...], preferred_element_type=f32) * relv[...]
    p0d[...] = jnp.dot(xd[...], ws0d[...], preferred_element_type=f32)


def _encode(xc, xd, wr00, ws0c, ws0d, wr10, ws1c, relv):
    row = lambda i: (i, 0)
    full = lambda i: (0, 0)
    return pl.pallas_call(
        _encode_body,
        grid=(NROW // BLK,),
        in_specs=[
            pl.BlockSpec((BLK, D), row),
            pl.BlockSpec((BLK, D), row),
            pl.BlockSpec((D, D), full),
            pl.BlockSpec((D, D), full),
            pl.BlockSpec((D, D), full),
            pl.BlockSpec((D, O), full),
            pl.BlockSpec((D, O), full),
            pl.BlockSpec((1, O), full),
        ],
        out_specs=[
            pl.BlockSpec((BLK, D), row),
            pl.BlockSpec((BLK, D), row),
            pl.BlockSpec((BLK, O), row),
            pl.BlockSpec((BLK, D), row),
        ],
        out_shape=[
            jax.ShapeDtypeStruct((NROW, D), jnp.float32),
            jax.ShapeDtypeStruct((NROW, D), jnp.float32),
            jax.ShapeDtypeStruct((NROW, O), jnp.float32),
            jax.ShapeDtypeStruct((NROW, D), jnp.float32),
        ],
    )(xc, xd, wr00, ws0c, ws0d, wr10, ws1c, relv)


def _segsum_body(y0_hbm, y1_hbm, src_hbm, dst_hbm, z_hbm,
                 agg0_out, agg1_out,
                 src_v, dst_v, r0, r1,
                 agg_s, sem0, sem1):
    cid = lax.axis_index("c")
    sid = lax.axis_index("s")
    rbase = sid * BLK

    pltpu.sync_copy(z_hbm.at[pl.ds(rbase, BLK)], agg_s.at[pl.ds(rbase, BLK)])
    plsc.subcore_barrier()

    pltpu.sync_copy(src_hbm.at[sid], src_v)
    pltpu.sync_copy(dst_hbm.at[sid], dst_v)

    def run(kc, coff):
        def phase(y_hbm, agg_out):
            def start(c, rv, sem):
                pltpu.async_copy(y_hbm.at[src_v.at[coff + c]], rv, sem)

            def wait(c, rv, sem):
                pltpu.make_async_copy(
                    y_hbm.at[src_v.at[coff + c]], rv, sem).wait()

            def scat(c, rv):
                pltpu.sync_copy(rv, agg_s.at[dst_v.at[coff + c]], add=True)

            start(0, r0, sem0)

            def kstep(k, _):
                e = 2 * k
                o = e + 1
                start(o, r1, sem1)
                wait(e, r0, sem0)
                scat(e, r0)
                start(e + 2, r0, sem0)
                wait(o, r1, sem1)
                scat(o, r1)
                return _

            lax.fori_loop(0, (kc - 2) // 2, kstep, None)
            start(kc - 1, r1, sem1)
            wait(kc - 2, r0, sem0)
            scat(kc - 2, r0)
            wait(kc - 1, r1, sem1)
            scat(kc - 1, r1)
            plsc.subcore_barrier()
            pltpu.sync_copy(agg_s.at[pl.ds(rbase, BLK)],
                            agg_out.at[cid, pl.ds(rbase, BLK)])

        phase(y0_hbm, agg0_out)
        pltpu.sync_copy(z_hbm.at[pl.ds(rbase, BLK)],
                        agg_s.at[pl.ds(rbase, BLK)])
        plsc.subcore_barrier()
        phase(y1_hbm, agg1_out)

    @pl.when(cid == 0)
    def _core0():
        run(EC0, 0)

    @pl.when(cid == 1)
    def _core1():
        run(EC1, EC0)


def _segsum(y0, y1a, srcb, dstb, zblk):
    mesh = plsc.VectorSubcoreMesh(core_axis_name="c", subcore_axis_name="s")
    fn = pl.kernel(
        _segsum_body,
        out_type=[
            jax.ShapeDtypeStruct((2, NROW, D), jnp.float32),
            jax.ShapeDtypeStruct((2, NROW, D), jnp.float32),
        ],
        mesh=mesh,
        scratch_types=[
            pltpu.VMEM((ECT, CB), jnp.int32),
            pltpu.VMEM((ECT, CB), jnp.int32),
            pltpu.VMEM((CB, D), jnp.float32),
            pltpu.VMEM((CB, D), jnp.float32),
            pltpu.VMEM_SHARED((NROW, D), jnp.float32),
            pltpu.SemaphoreType.DMA,
            pltpu.SemaphoreType.DMA,
        ],
    )
    return fn(y0, y1a, srcb, dstb, zblk)


def _combine_body(p0d, a0a, a0b, a1a, a1b, ws1d, hd):
    col = lax.broadcasted_iota(jnp.int32, (BLK, D), 1)
    a0 = a0a[...] + a0b[...]
    a1 = a1a[...] + a1b[...]
    deg = jnp.sum(jnp.where(col == O, a1, 0.0), axis=1, keepdims=True)
    dd = jnp.maximum(deg, 1.0)
    h1d = jnp.maximum(p0d[...] + a0 / dd, 0.0)
    hd[...] = (jnp.dot(h1d, ws1d[...], preferred_element_type=jnp.float32)
               + a1[:, :O] / dd)


def _combine(p0d, a0a, a0b, a1a, a1b, ws1d):
    row = lambda i: (i, 0)
    full = lambda i: (0, 0)
    return pl.pallas_call(
        _combine_body,
        grid=(NROW // BLK,),
        in_specs=[
            pl.BlockSpec((BLK, D), row),
            pl.BlockSpec((BLK, D), row),
            pl.BlockSpec((BLK, D), row),
            pl.BlockSpec((BLK, D), row),
            pl.BlockSpec((BLK, D), row),
            pl.BlockSpec((D, O), full),
        ],
        out_specs=pl.BlockSpec((BLK, O), row),
        out_shape=jax.ShapeDtypeStruct((NROW, O), jnp.float32),
    )(p0d, a0a, a0b, a1a, a1b, ws1d)


def _decode_body(gc_hbm, hd_hbm, aidx_hbm, bidx_hbm, out_hbm,
                 aidx_v, bidx_v, gav, gbv, sc_v,
                 sav, sbv):
    cid = lax.axis_index("c")
    sid = lax.axis_index("s")
    wid = sid * 2 + cid
    pltpu.sync_copy(aidx_hbm.at[wid], aidx_v)
    pltpu.sync_copy(bidx_hbm.at[wid], bidx_v)

    lanes = lax.iota(jnp.int32, 16)

    def compute(cc, ga, gb):
        def group(g, _):
            pv = g * 16 + lanes
            acc = jnp.zeros((16,), jnp.float32)
            for j in range(O):
                jf = (lanes + j) & (O - 1)
                va = plsc.load_gather(ga, [pv, jf])
                vb = plsc.load_gather(gb, [pv, jf])
                acc = acc + va * vb
            sc_v[cc, pl.ds(g * 16, 16)] = acc
            return _

        lax.fori_loop(0, CB // 16, group, None)

    NB = 4

    def start(cc, b):
        pltpu.async_copy(gc_hbm.at[aidx_v.at[cc]], gav.at[b], sav.at[b])
        pltpu.async_copy(hd_hbm.at[bidx_v.at[cc]], gbv.at[b], sbv.at[b])

    def wait(cc, b):
        pltpu.make_async_copy(
            gc_hbm.at[aidx_v.at[cc]], gav.at[b], sav.at[b]).wait()
        pltpu.make_async_copy(
            hd_hbm.at[bidx_v.at[cc]], gbv.at[b], sbv.at[b]).wait()

    for b in range(NB):
        start(b, b)

    def kstep(k, _):
        for b in range(NB):
            cc = NB * k + b
            wait(cc, b)
            compute(cc, gav.at[b], gbv.at[b])

            @pl.when(cc + NB < PC)
            def _prefetch():
                start(cc + NB, b)

        return _

    lax.fori_loop(0, PC // NB, kstep, None)
    wait(PC - 1, (PC - 1) % NB)
    compute(PC - 1, gav.at[(PC - 1) % NB], gbv.at[(PC - 1) % NB])
    pltpu.sync_copy(sc_v, out_hbm.at[wid])


def _decode(gc, hd, aidxb, bidxb):
    mesh = plsc.VectorSubcoreMesh(core_axis_name="c", subcore_axis_name="s")
    fn = pl.kernel(
        _decode_body,
        out_type=jax.ShapeDtypeStruct((NW, PC, CB), jnp.float32),
        mesh=mesh,
        compiler_params=_SC_PARAMS,
        scratch_types=[
            pltpu.VMEM((PC, CB), jnp.int32),
            pltpu.VMEM((PC, CB), jnp.int32),
            pltpu.VMEM((4, CB, O), jnp.float32),
            pltpu.VMEM((4, CB, O), jnp.float32),
            pltpu.VMEM((PC, CB), jnp.float32),
            pltpu.SemaphoreType.DMA((4,)),
            pltpu.SemaphoreType.DMA((4,)),
        ],
    )
    return fn(gc, hd, aidxb, bidxb)


def kernel(x_Compound, x_Disease, x_Gene, x_Anatomy,
           edge_index_0, edge_index_1, edge_index_2, edge_index_3,
           W_self_0_Compound, W_self_0_Disease, W_self_0_Gene, W_self_0_Anatomy,
           W_rel_0_0, W_rel_0_1, W_rel_0_2, W_rel_0_3,
           W_self_1_Compound, W_self_1_Disease, W_self_1_Gene, W_self_1_Anatomy,
           W_rel_1_0, W_rel_1_1, W_rel_1_2, W_rel_1_3,
           rel_vec, edge_label_index):
    f32 = jnp.float32
    xc = jnp.pad(x_Compound.astype(f32), ((0, NROW - N), (0, 0)))
    xd = jnp.pad(x_Disease.astype(f32), ((0, NROW - N), (0, 0)))

    src = edge_index_0[0].astype(jnp.int32)
    dst = edge_index_0[1].astype(jnp.int32)
    epad = EPAD - E
    srcb = jnp.concatenate([src, jnp.zeros((epad,), jnp.int32)]).reshape(16, ECT, CB)
    dstb = jnp.concatenate([dst, jnp.full((epad,), N, jnp.int32)]).reshape(16, ECT, CB)

    eli = edge_label_index.astype(jnp.int32)
    ppad = NW * P_PER - P
    aidxb = jnp.concatenate([eli[0], jnp.zeros((ppad,), jnp.int32)]).reshape(NW, PC, CB)
    bidxb = jnp.concatenate([eli[1], jnp.zeros((ppad,), jnp.int32)]).reshape(NW, PC, CB)

    z128 = jnp.zeros((NROW, D), f32)

    y0, y1a, gc, p0d = _encode(
        xc, xd, W_rel_0_0, W_self_0_Compound, W_self_0_Disease,
        W_rel_1_0, W_self_1_Compound, rel_vec.reshape(1, O))

    agg0p, agg1p = _segsum(y0, y1a, srcb, dstb, z128)

    hd = _combine(p0d, agg0p[0], agg0p[1], agg1p[0], agg1p[1],
                  W_self_1_Disease)

    scores = _decode(gc, hd, aidxb, bidxb)
    return scores.reshape(-1)[:P]
